# Initial kernel scaffold; baseline (speedup 1.0000x reference)
import jax
import jax.numpy as jnp
from jax.experimental import pallas as pl


def kernel(node_memory, node_features, edge_features, time_encoding, edge_index_causal, edge_index_conseq, W_msg, W_upd, U_upd, b_upd):
    raise NotImplementedError("write your pallas kernel here")


# speedup vs baseline: 2.5086x; 2.5086x over previous
import functools

import jax
import jax.numpy as jnp
from jax import lax
from jax.experimental import pallas as pl
from jax.experimental.pallas import tpu as pltpu
from jax.experimental.pallas import tpu_sc as plsc

N_NODES = 10000
N_EDGES = 320000
DM = 128
DH = 64

NC = 2
NS = 16
NW = NC * NS
EDGES_PER_W = N_EDGES // NW
CHUNK = 80
STEPS = EDGES_PER_W // CHUNK
N_PAD = 10240
ROWS_PER_TILE = N_PAD // NS


def _edge_term_body(x_ref, w_ref, o_ref):
    r = jnp.dot(x_ref[...], w_ref[...], preferred_element_type=jnp.float32)
    o_ref[0] = r[:, :DH]
    o_ref[1] = r[:, DH:]


def _edge_term(x, w):
    blk = 6400
    grid = N_EDGES // blk
    return pl.pallas_call(
        _edge_term_body,
        grid=(grid,),
        in_specs=[
            pl.BlockSpec((blk, 32), lambda i: (i, 0)),
            pl.BlockSpec((32, DM), lambda i: (0, 0)),
        ],
        out_specs=pl.BlockSpec((2, blk, DH), lambda i: (0, i, 0)),
        out_shape=jax.ShapeDtypeStruct((2, N_EDGES, DH), jnp.float32),
    )(x, w)


def _node_term_body(mem_ref, wm_ref, q_ref, o_ref):
    r = (
        jnp.dot(mem_ref[...], wm_ref[...], preferred_element_type=jnp.float32)
        + q_ref[...]
    )
    o_ref[0] = r[:, :DH]
    o_ref[1] = r[:, DH:]


def _node_term(mem, wm, q):
    return pl.pallas_call(
        _node_term_body,
        grid=(1,),
        in_specs=[
            pl.BlockSpec((N_NODES, DM), lambda i: (0, 0)),
            pl.BlockSpec((DM, DM), lambda i: (0, 0)),
            pl.BlockSpec((N_NODES, DM), lambda i: (0, 0)),
        ],
        out_specs=pl.BlockSpec((2, N_NODES, DH), lambda i: (0, 0, 0)),
        out_shape=jax.ShapeDtypeStruct((2, N_NODES, DH), jnp.float32),
    )(mem, wm, q)


def _q_body(nf_ref, wf_ref, o_ref):
    o_ref[...] = jnp.dot(nf_ref[...], wf_ref[...], preferred_element_type=jnp.float32)


def _q_term(nf, wf):
    return pl.pallas_call(
        _q_body,
        grid=(1,),
        in_specs=[
            pl.BlockSpec((N_NODES, DM), lambda i: (0, 0)),
            pl.BlockSpec((DM, DM), lambda i: (0, 0)),
        ],
        out_specs=pl.BlockSpec((N_NODES, DM), lambda i: (0, 0)),
        out_shape=jax.ShapeDtypeStruct((N_NODES, DM), jnp.float32),
    )(nf, wf)


def _update_body(agga_ref, aggb_ref, mem_ref, wu_ref, uu_ref, b_ref, m1_ref,
                 o_ref, *, mean_with_m1):
    ha = agga_ref[0, :N_NODES] + agga_ref[1, :N_NODES]
    hb = aggb_ref[0, :N_NODES] + aggb_ref[1, :N_NODES]
    agg = jnp.concatenate([ha, hb], axis=-1)
    mem2 = jnp.tanh(
        jnp.dot(agg, wu_ref[...], preferred_element_type=jnp.float32)
        + jnp.dot(mem_ref[...], uu_ref[...], preferred_element_type=jnp.float32)
        + b_ref[...]
    )
    if mean_with_m1:
        o_ref[...] = 0.5 * (m1_ref[...] + mem2)
    else:
        o_ref[...] = mem2


def _update(agga, aggb, mem, wu, uu, b, m1, mean_with_m1):
    return pl.pallas_call(
        functools.partial(_update_body, mean_with_m1=mean_with_m1),
        grid=(1,),
        in_specs=[
            pl.BlockSpec((NC, N_PAD, DH), lambda i: (0, 0, 0)),
            pl.BlockSpec((NC, N_PAD, DH), lambda i: (0, 0, 0)),
            pl.BlockSpec((N_NODES, DM), lambda i: (0, 0)),
            pl.BlockSpec((DM, DM), lambda i: (0, 0)),
            pl.BlockSpec((DM, DM), lambda i: (0, 0)),
            pl.BlockSpec((1, DM), lambda i: (0, 0)),
            pl.BlockSpec((N_NODES, DM), lambda i: (0, 0)),
        ],
        out_specs=pl.BlockSpec((N_NODES, DM), lambda i: (0, 0)),
        out_shape=jax.ShapeDtypeStruct((N_NODES, DM), jnp.float32),
    )(agga, aggb, mem, wu, uu, b, m1)


def _sc_body(p_hbm, e_hbm, src_hbm, dst_hbm, zeros_hbm, out_hbm,
             src_v, dst_v, rows_v, e_v, agg_sh, sem):
    c = lax.axis_index("c")
    s = lax.axis_index("s")
    wid = c * NS + s

    pltpu.sync_copy(zeros_hbm, agg_sh.at[pl.ds(s * ROWS_PER_TILE, ROWS_PER_TILE)])
    plsc.subcore_barrier()

    base_edge = wid * EDGES_PER_W
    pltpu.sync_copy(src_hbm.at[wid], src_v)
    pltpu.sync_copy(dst_hbm.at[wid], dst_v)

    def step(j, carry):
        pltpu.async_copy(p_hbm.at[src_v.at[j]], rows_v, sem).wait()
        pltpu.sync_copy(e_hbm.at[pl.ds(base_edge + j * CHUNK, CHUNK)], e_v)

        def erow(r, carry2):
            for k in range(DH // 16):
                sl = pl.ds(k * 16, 16)
                rows_v[r, sl] = jnp.maximum(rows_v[r, sl] + e_v[r, sl], 0.0)
            return carry2

        lax.fori_loop(0, CHUNK, erow, 0)

        pltpu.sync_copy(rows_v, agg_sh.at[dst_v.at[j]], add=True)
        return carry

    lax.fori_loop(0, STEPS, step, 0)
    plsc.subcore_barrier()

    pltpu.sync_copy(
        agg_sh.at[pl.ds(s * ROWS_PER_TILE, ROWS_PER_TILE)],
        out_hbm.at[c, pl.ds(s * ROWS_PER_TILE, ROWS_PER_TILE)],
    )


_sc_edge_pass = functools.partial(
    pl.kernel,
    out_type=jax.ShapeDtypeStruct((NC, N_PAD, DH), jnp.float32),
    mesh=plsc.VectorSubcoreMesh(core_axis_name="c", subcore_axis_name="s",
                                num_cores=NC, num_subcores=NS),
    scratch_types=[
        pltpu.VMEM((STEPS, CHUNK), jnp.int32),
        pltpu.VMEM((STEPS, CHUNK), jnp.int32),
        pltpu.VMEM((CHUNK, DH), jnp.float32),
        pltpu.VMEM((CHUNK, DH), jnp.float32),
        pltpu.VMEM_SHARED((N_PAD, DH), jnp.float32),
        pltpu.SemaphoreType.DMA,
    ],
    compiler_params=pltpu.CompilerParams(use_tc_tiling_on_sc=False),
)(_sc_body)


def kernel(node_memory, node_features, edge_features, time_encoding,
           edge_index_causal, edge_index_conseq,
           W_msg, W_upd, U_upd, b_upd):
    wm = W_msg[:DM]
    wf = W_msg[DM:2 * DM]
    wet = W_msg[2 * DM:]

    et = jnp.concatenate([edge_features, time_encoding], axis=1)
    e_term = _edge_term(et, wet)
    q = _q_term(node_features, wf)

    zeros = jnp.zeros((ROWS_PER_TILE, DH), jnp.float32)

    b2 = b_upd.reshape(1, DM)

    src1 = edge_index_causal[0].astype(jnp.int32).reshape(NW, STEPS, CHUNK)
    dst1 = edge_index_causal[1].astype(jnp.int32).reshape(NW, STEPS, CHUNK)
    src2 = edge_index_conseq[0].astype(jnp.int32).reshape(NW, STEPS, CHUNK)
    dst2 = edge_index_conseq[1].astype(jnp.int32).reshape(NW, STEPS, CHUNK)

    p1 = _node_term(node_memory, wm, q)
    agg1a = _sc_edge_pass(p1[0], e_term[0], src1, dst1, zeros)
    agg1b = _sc_edge_pass(p1[1], e_term[1], src1, dst1, zeros)
    mem1 = _update(agg1a, agg1b, node_memory, W_upd, U_upd, b2, node_memory, False)

    p2 = _node_term(mem1, wm, q)
    agg2a = _sc_edge_pass(p2[0], e_term[0], src2, dst2, zeros)
    agg2b = _sc_edge_pass(p2[1], e_term[1], src2, dst2, zeros)
    return _update(agg2a, agg2b, mem1, W_upd, U_upd, b2, mem1, True)


# speedup vs baseline: 4.6849x; 1.8675x over previous
import functools

import jax
import jax.numpy as jnp
from jax import lax
from jax.experimental import pallas as pl
from jax.experimental.pallas import tpu as pltpu
from jax.experimental.pallas import tpu_sc as plsc

N_NODES = 10000
N_EDGES = 320000
DM = 128
DH = 64

NC = 2
NS = 16
NW = NC * NS
EDGES_PER_W = N_EDGES // NW
CHUNK = 125
STEPS = EDGES_PER_W // CHUNK
NB = 4
GROUPS = STEPS // NB
N_PAD = 10240
ROWS_PER_TILE = N_PAD // NS


def _edge_term_body(x_ref, w_ref, oa_ref, ob_ref):
    r = jnp.dot(x_ref[...], w_ref[...], preferred_element_type=jnp.float32)
    oa_ref[...] = r[:, :DH]
    ob_ref[...] = r[:, DH:]


def _edge_term(x, w):
    blk = 6400
    grid = N_EDGES // blk
    return pl.pallas_call(
        _edge_term_body,
        grid=(grid,),
        in_specs=[
            pl.BlockSpec((blk, 32), lambda i: (i, 0)),
            pl.BlockSpec((32, DM), lambda i: (0, 0)),
        ],
        out_specs=[
            pl.BlockSpec((blk, DH), lambda i: (i, 0)),
            pl.BlockSpec((blk, DH), lambda i: (i, 0)),
        ],
        out_shape=[
            jax.ShapeDtypeStruct((N_EDGES, DH), jnp.float32),
            jax.ShapeDtypeStruct((N_EDGES, DH), jnp.float32),
        ],
    )(x, w)


def _node_term_body(mem_ref, wm_ref, q_ref, oa_ref, ob_ref):
    r = (
        jnp.dot(mem_ref[...], wm_ref[...], preferred_element_type=jnp.float32)
        + q_ref[...]
    )
    oa_ref[...] = r[:, :DH]
    ob_ref[...] = r[:, DH:]


def _node_term(mem, wm, q):
    return pl.pallas_call(
        _node_term_body,
        grid=(1,),
        in_specs=[
            pl.BlockSpec((N_NODES, DM), lambda i: (0, 0)),
            pl.BlockSpec((DM, DM), lambda i: (0, 0)),
            pl.BlockSpec((N_NODES, DM), lambda i: (0, 0)),
        ],
        out_specs=[
            pl.BlockSpec((N_NODES, DH), lambda i: (0, 0)),
            pl.BlockSpec((N_NODES, DH), lambda i: (0, 0)),
        ],
        out_shape=[
            jax.ShapeDtypeStruct((N_NODES, DH), jnp.float32),
            jax.ShapeDtypeStruct((N_NODES, DH), jnp.float32),
        ],
    )(mem, wm, q)


def _q_body(nf_ref, wf_ref, o_ref):
    o_ref[...] = jnp.dot(nf_ref[...], wf_ref[...], preferred_element_type=jnp.float32)


def _q_term(nf, wf):
    return pl.pallas_call(
        _q_body,
        grid=(1,),
        in_specs=[
            pl.BlockSpec((N_NODES, DM), lambda i: (0, 0)),
            pl.BlockSpec((DM, DM), lambda i: (0, 0)),
        ],
        out_specs=pl.BlockSpec((N_NODES, DM), lambda i: (0, 0)),
        out_shape=jax.ShapeDtypeStruct((N_NODES, DM), jnp.float32),
    )(nf, wf)


def _update_body(agga_ref, aggb_ref, mem_ref, wu_ref, uu_ref, b_ref, m1_ref,
                 o_ref, *, mean_with_m1):
    ha = agga_ref[0, :N_NODES] + agga_ref[1, :N_NODES]
    hb = aggb_ref[0, :N_NODES] + aggb_ref[1, :N_NODES]
    agg = jnp.concatenate([ha, hb], axis=-1)
    mem2 = jnp.tanh(
        jnp.dot(agg, wu_ref[...], preferred_element_type=jnp.float32)
        + jnp.dot(mem_ref[...], uu_ref[...], preferred_element_type=jnp.float32)
        + b_ref[...]
    )
    if mean_with_m1:
        o_ref[...] = 0.5 * (m1_ref[...] + mem2)
    else:
        o_ref[...] = mem2


def _update(agga, aggb, mem, wu, uu, b, m1, mean_with_m1):
    return pl.pallas_call(
        functools.partial(_update_body, mean_with_m1=mean_with_m1),
        grid=(1,),
        in_specs=[
            pl.BlockSpec((NC, N_PAD, DH), lambda i: (0, 0, 0)),
            pl.BlockSpec((NC, N_PAD, DH), lambda i: (0, 0, 0)),
            pl.BlockSpec((N_NODES, DM), lambda i: (0, 0)),
            pl.BlockSpec((DM, DM), lambda i: (0, 0)),
            pl.BlockSpec((DM, DM), lambda i: (0, 0)),
            pl.BlockSpec((1, DM), lambda i: (0, 0)),
            pl.BlockSpec((N_NODES, DM), lambda i: (0, 0)),
        ],
        out_specs=pl.BlockSpec((N_NODES, DM), lambda i: (0, 0)),
        out_shape=jax.ShapeDtypeStruct((N_NODES, DM), jnp.float32),
    )(agga, aggb, mem, wu, uu, b, m1)


def _sc_body(p_hbm, e_hbm, src_hbm, dst_hbm, zeros_hbm, out_hbm, *refs):
    rows = refs[0:NB]
    evs = refs[NB:2 * NB]
    src_v = refs[2 * NB]
    dst_v = refs[2 * NB + 1]
    agg_sh = refs[2 * NB + 2]
    g_sems = refs[2 * NB + 3:2 * NB + 3 + NB]
    e_sems = refs[2 * NB + 3 + NB:2 * NB + 3 + 2 * NB]
    s_sems = refs[2 * NB + 3 + 2 * NB:2 * NB + 3 + 3 * NB]

    c = lax.axis_index("c")
    s = lax.axis_index("s")
    wid = c * NS + s

    pltpu.sync_copy(zeros_hbm, agg_sh.at[pl.ds(s * ROWS_PER_TILE, ROWS_PER_TILE)])
    plsc.subcore_barrier()

    base_edge = wid * EDGES_PER_W
    pltpu.sync_copy(src_hbm.at[wid], src_v)
    pltpu.sync_copy(dst_hbm.at[wid], dst_v)

    def issue(g, b):
        j = g * NB + b
        pltpu.async_copy(p_hbm.at[src_v.at[j]], rows[b], g_sems[b])
        pltpu.async_copy(
            e_hbm.at[pl.ds(base_edge + j * CHUNK, CHUNK)], evs[b], e_sems[b])

    for b in range(NB):
        issue(0, b)

    def group(g, carry):
        for b in range(NB):
            j = g * NB + b
            pltpu.make_async_copy(p_hbm.at[src_v.at[j]], rows[b], g_sems[b]).wait()
            pltpu.make_async_copy(
                e_hbm.at[pl.ds(base_edge + j * CHUNK, CHUNK)], evs[b],
                e_sems[b]).wait()

            def erow(r, carry2, b=b):
                for k in range(DH // 16):
                    sl = pl.ds(k * 16, 16)
                    rows[b][r, sl] = jnp.maximum(rows[b][r, sl] + evs[b][r, sl], 0.0)
                return carry2

            lax.fori_loop(0, CHUNK, erow, 0)

            pltpu.async_copy(rows[b], agg_sh.at[dst_v.at[j]], s_sems[b], add=True)

        @pl.when(g < GROUPS - 1)
        def _prefetch():
            for b in range(NB):
                j = g * NB + b
                pltpu.make_async_copy(
                    rows[b], agg_sh.at[dst_v.at[j]], s_sems[b]).wait()
                issue(g + 1, b)

        return carry

    lax.fori_loop(0, GROUPS, group, 0)

    for b in range(NB):
        j = (GROUPS - 1) * NB + b
        pltpu.make_async_copy(rows[b], agg_sh.at[dst_v.at[j]], s_sems[b]).wait()

    plsc.subcore_barrier()

    pltpu.sync_copy(
        agg_sh.at[pl.ds(s * ROWS_PER_TILE, ROWS_PER_TILE)],
        out_hbm.at[c, pl.ds(s * ROWS_PER_TILE, ROWS_PER_TILE)],
    )


_sc_edge_pass = functools.partial(
    pl.kernel,
    out_type=jax.ShapeDtypeStruct((NC, N_PAD, DH), jnp.float32),
    mesh=plsc.VectorSubcoreMesh(core_axis_name="c", subcore_axis_name="s",
                                num_cores=NC, num_subcores=NS),
    scratch_types=(
        [pltpu.VMEM((CHUNK, DH), jnp.float32) for _ in range(2 * NB)]
        + [
            pltpu.VMEM((STEPS, CHUNK), jnp.int32),
            pltpu.VMEM((STEPS, CHUNK), jnp.int32),
            pltpu.VMEM_SHARED((N_PAD, DH), jnp.float32),
        ]
        + [pltpu.SemaphoreType.DMA for _ in range(3 * NB)]
    ),
    compiler_params=pltpu.CompilerParams(use_tc_tiling_on_sc=False),
)(_sc_body)


def kernel(node_memory, node_features, edge_features, time_encoding,
           edge_index_causal, edge_index_conseq,
           W_msg, W_upd, U_upd, b_upd):
    wm = W_msg[:DM]
    wf = W_msg[DM:2 * DM]
    wet = W_msg[2 * DM:]

    et = jnp.concatenate([edge_features, time_encoding], axis=1)
    ea, eb = _edge_term(et, wet)
    q = _q_term(node_features, wf)

    zeros = jnp.zeros((ROWS_PER_TILE, DH), jnp.float32)

    b2 = b_upd.reshape(1, DM)

    src1 = edge_index_causal[0].astype(jnp.int32).reshape(NW, STEPS, CHUNK)
    dst1 = edge_index_causal[1].astype(jnp.int32).reshape(NW, STEPS, CHUNK)
    src2 = edge_index_conseq[0].astype(jnp.int32).reshape(NW, STEPS, CHUNK)
    dst2 = edge_index_conseq[1].astype(jnp.int32).reshape(NW, STEPS, CHUNK)

    p1a, p1b = _node_term(node_memory, wm, q)
    agg1a = _sc_edge_pass(p1a, ea, src1, dst1, zeros)
    agg1b = _sc_edge_pass(p1b, eb, src1, dst1, zeros)
    mem1 = _update(agg1a, agg1b, node_memory, W_upd, U_upd, b2, node_memory, False)

    p2a, p2b = _node_term(mem1, wm, q)
    agg2a = _sc_edge_pass(p2a, ea, src2, dst2, zeros)
    agg2b = _sc_edge_pass(p2b, eb, src2, dst2, zeros)
    return _update(agg2a, agg2b, mem1, W_upd, U_upd, b2, mem1, True)


# speedup vs baseline: 5.2462x; 1.1198x over previous
import functools

import jax
import jax.numpy as jnp
from jax import lax
from jax.experimental import pallas as pl
from jax.experimental.pallas import tpu as pltpu
from jax.experimental.pallas import tpu_sc as plsc

N_NODES = 10000
N_EDGES = 320000
DM = 128
DH = 64

NC = 2
NS = 16
NW = NC * NS
EDGES_PER_W = N_EDGES // NW
CHUNK = 125
STEPS = EDGES_PER_W // CHUNK
NB = 4
GROUPS = STEPS // NB
N_PAD = 10240
ROWS_PER_TILE = N_PAD // NS


def _edge_term_body(ef_ref, te_ref, we_ref, wt_ref, o_ref):
    o_ref[...] = (
        jnp.dot(ef_ref[...], we_ref[...], preferred_element_type=jnp.float32)
        + jnp.dot(te_ref[...], wt_ref[...], preferred_element_type=jnp.float32)
    )


def _edge_term(ef, te, we, wt):
    blk = 6400
    grid = N_EDGES // blk
    return pl.pallas_call(
        _edge_term_body,
        grid=(grid,),
        in_specs=[
            pl.BlockSpec((blk, 16), lambda i: (i, 0)),
            pl.BlockSpec((blk, 16), lambda i: (i, 0)),
            pl.BlockSpec((16, DM), lambda i: (0, 0)),
            pl.BlockSpec((16, DM), lambda i: (0, 0)),
        ],
        out_specs=pl.BlockSpec((blk, DM), lambda i: (i, 0)),
        out_shape=jax.ShapeDtypeStruct((N_EDGES, DM), jnp.float32),
    )(ef, te, we, wt)


def _node_term_body(mem_ref, wm_ref, q_ref, oa_ref, ob_ref):
    r = (
        jnp.dot(mem_ref[...], wm_ref[...], preferred_element_type=jnp.float32)
        + q_ref[...]
    )
    oa_ref[...] = r[:, :DH]
    ob_ref[...] = r[:, DH:]


def _node_term(mem, wm, q):
    return pl.pallas_call(
        _node_term_body,
        grid=(1,),
        in_specs=[
            pl.BlockSpec((N_NODES, DM), lambda i: (0, 0)),
            pl.BlockSpec((DM, DM), lambda i: (0, 0)),
            pl.BlockSpec((N_NODES, DM), lambda i: (0, 0)),
        ],
        out_specs=[
            pl.BlockSpec((N_NODES, DH), lambda i: (0, 0)),
            pl.BlockSpec((N_NODES, DH), lambda i: (0, 0)),
        ],
        out_shape=[
            jax.ShapeDtypeStruct((N_NODES, DH), jnp.float32),
            jax.ShapeDtypeStruct((N_NODES, DH), jnp.float32),
        ],
    )(mem, wm, q)


def _q_body(nf_ref, wf_ref, o_ref):
    o_ref[...] = jnp.dot(nf_ref[...], wf_ref[...], preferred_element_type=jnp.float32)


def _q_term(nf, wf):
    return pl.pallas_call(
        _q_body,
        grid=(1,),
        in_specs=[
            pl.BlockSpec((N_NODES, DM), lambda i: (0, 0)),
            pl.BlockSpec((DM, DM), lambda i: (0, 0)),
        ],
        out_specs=pl.BlockSpec((N_NODES, DM), lambda i: (0, 0)),
        out_shape=jax.ShapeDtypeStruct((N_NODES, DM), jnp.float32),
    )(nf, wf)


def _update_body(agga_ref, aggb_ref, mem_ref, wu_ref, uu_ref, b_ref, m1_ref,
                 o_ref, *, mean_with_m1):
    ha = agga_ref[0, :N_NODES] + agga_ref[1, :N_NODES]
    hb = aggb_ref[0, :N_NODES] + aggb_ref[1, :N_NODES]
    agg = jnp.concatenate([ha, hb], axis=-1)
    mem2 = jnp.tanh(
        jnp.dot(agg, wu_ref[...], preferred_element_type=jnp.float32)
        + jnp.dot(mem_ref[...], uu_ref[...], preferred_element_type=jnp.float32)
        + b_ref[...]
    )
    if mean_with_m1:
        o_ref[...] = 0.5 * (m1_ref[...] + mem2)
    else:
        o_ref[...] = mem2


def _update(agga, aggb, mem, wu, uu, b, m1, mean_with_m1):
    return pl.pallas_call(
        functools.partial(_update_body, mean_with_m1=mean_with_m1),
        grid=(1,),
        in_specs=[
            pl.BlockSpec((NC, N_PAD, DH), lambda i: (0, 0, 0)),
            pl.BlockSpec((NC, N_PAD, DH), lambda i: (0, 0, 0)),
            pl.BlockSpec((N_NODES, DM), lambda i: (0, 0)),
            pl.BlockSpec((DM, DM), lambda i: (0, 0)),
            pl.BlockSpec((DM, DM), lambda i: (0, 0)),
            pl.BlockSpec((1, DM), lambda i: (0, 0)),
            pl.BlockSpec((N_NODES, DM), lambda i: (0, 0)),
        ],
        out_specs=pl.BlockSpec((N_NODES, DM), lambda i: (0, 0)),
        out_shape=jax.ShapeDtypeStruct((N_NODES, DM), jnp.float32),
    )(agga, aggb, mem, wu, uu, b, m1)


def _sc_body(p_hbm, e_hbm, src_hbm, dst_hbm, zeros_hbm, out_hbm, *refs, hoff):
    rows = refs[0:NB]
    evs = refs[NB:2 * NB]
    src_v = refs[2 * NB]
    dst_v = refs[2 * NB + 1]
    agg_sh = refs[2 * NB + 2]
    g_sems = refs[2 * NB + 3:2 * NB + 3 + NB]
    e_sems = refs[2 * NB + 3 + NB:2 * NB + 3 + 2 * NB]
    s_sems = refs[2 * NB + 3 + 2 * NB:2 * NB + 3 + 3 * NB]

    c = lax.axis_index("c")
    s = lax.axis_index("s")
    wid = c * NS + s

    pltpu.sync_copy(zeros_hbm, agg_sh.at[pl.ds(s * ROWS_PER_TILE, ROWS_PER_TILE)])
    plsc.subcore_barrier()

    base_edge = wid * EDGES_PER_W
    pltpu.sync_copy(src_hbm.at[wid], src_v)
    pltpu.sync_copy(dst_hbm.at[wid], dst_v)

    def issue(g, b):
        j = g * NB + b
        pltpu.async_copy(p_hbm.at[src_v.at[j]], rows[b], g_sems[b])
        pltpu.async_copy(
            e_hbm.at[pl.ds(base_edge + j * CHUNK, CHUNK), pl.ds(hoff, DH)],
            evs[b], e_sems[b])

    for b in range(NB):
        issue(0, b)

    def group(g, carry):
        for b in range(NB):
            j = g * NB + b
            pltpu.make_async_copy(p_hbm.at[src_v.at[j]], rows[b], g_sems[b]).wait()
            pltpu.make_async_copy(
                e_hbm.at[pl.ds(base_edge + j * CHUNK, CHUNK), pl.ds(hoff, DH)],
                evs[b], e_sems[b]).wait()

            def erow(r, carry2, b=b):
                for k in range(DH // 16):
                    sl = pl.ds(k * 16, 16)
                    rows[b][r, sl] = jnp.maximum(rows[b][r, sl] + evs[b][r, sl], 0.0)
                return carry2

            lax.fori_loop(0, CHUNK, erow, 0)

            pltpu.async_copy(rows[b], agg_sh.at[dst_v.at[j]], s_sems[b], add=True)

        @pl.when(g < GROUPS - 1)
        def _prefetch():
            for b in range(NB):
                j = g * NB + b
                pltpu.make_async_copy(
                    rows[b], agg_sh.at[dst_v.at[j]], s_sems[b]).wait()
                issue(g + 1, b)

        return carry

    lax.fori_loop(0, GROUPS, group, 0)

    for b in range(NB):
        j = (GROUPS - 1) * NB + b
        pltpu.make_async_copy(rows[b], agg_sh.at[dst_v.at[j]], s_sems[b]).wait()

    plsc.subcore_barrier()

    pltpu.sync_copy(
        agg_sh.at[pl.ds(s * ROWS_PER_TILE, ROWS_PER_TILE)],
        out_hbm.at[c, pl.ds(s * ROWS_PER_TILE, ROWS_PER_TILE)],
    )


def _make_sc_pass(hoff):
    return functools.partial(
        pl.kernel,
        out_type=jax.ShapeDtypeStruct((NC, N_PAD, DH), jnp.float32),
        mesh=plsc.VectorSubcoreMesh(core_axis_name="c", subcore_axis_name="s",
                                    num_cores=NC, num_subcores=NS),
        scratch_types=(
            [pltpu.VMEM((CHUNK, DH), jnp.float32) for _ in range(2 * NB)]
            + [
                pltpu.VMEM((STEPS, CHUNK), jnp.int32),
                pltpu.VMEM((STEPS, CHUNK), jnp.int32),
                pltpu.VMEM_SHARED((N_PAD, DH), jnp.float32),
            ]
            + [pltpu.SemaphoreType.DMA for _ in range(3 * NB)]
        ),
        compiler_params=pltpu.CompilerParams(use_tc_tiling_on_sc=False),
    )(functools.partial(_sc_body, hoff=hoff))


_sc_edge_pass_a = _make_sc_pass(0)
_sc_edge_pass_b = _make_sc_pass(DH)


def kernel(node_memory, node_features, edge_features, time_encoding,
           edge_index_causal, edge_index_conseq,
           W_msg, W_upd, U_upd, b_upd):
    wm = W_msg[:DM]
    wf = W_msg[DM:2 * DM]
    wet = W_msg[2 * DM:]

    we = wet[:16]
    wt = wet[16:]
    e_term = _edge_term(edge_features, time_encoding, we, wt)
    q = _q_term(node_features, wf)

    zeros = jnp.zeros((ROWS_PER_TILE, DH), jnp.float32)

    b2 = b_upd.reshape(1, DM)

    src1 = edge_index_causal[0].astype(jnp.int32).reshape(NW, STEPS, CHUNK)
    dst1 = edge_index_causal[1].astype(jnp.int32).reshape(NW, STEPS, CHUNK)
    src2 = edge_index_conseq[0].astype(jnp.int32).reshape(NW, STEPS, CHUNK)
    dst2 = edge_index_conseq[1].astype(jnp.int32).reshape(NW, STEPS, CHUNK)

    p1a, p1b = _node_term(node_memory, wm, q)
    agg1a = _sc_edge_pass_a(p1a, e_term, src1, dst1, zeros)
    agg1b = _sc_edge_pass_b(p1b, e_term, src1, dst1, zeros)
    mem1 = _update(agg1a, agg1b, node_memory, W_upd, U_upd, b2, node_memory, False)

    p2a, p2b = _node_term(mem1, wm, q)
    agg2a = _sc_edge_pass_a(p2a, e_term, src2, dst2, zeros)
    agg2b = _sc_edge_pass_b(p2b, e_term, src2, dst2, zeros)
    return _update(agg2a, agg2b, mem1, W_upd, U_upd, b2, mem1, True)


# speedup vs baseline: 6.8331x; 1.3025x over previous
import functools

import jax
import jax.numpy as jnp
from jax import lax
from jax.experimental import pallas as pl
from jax.experimental.pallas import tpu as pltpu
from jax.experimental.pallas import tpu_sc as plsc

N_NODES = 10000
N_EDGES = 320000
DM = 128
DH = 64

NC = 2
NS = 16
NW = NC * NS
EDGES_PER_W = N_EDGES // NW
CHUNK = 125
STEPS = EDGES_PER_W // CHUNK
NB = 4
GROUPS = STEPS // NB
N_PAD = 10240
ROWS_PER_TILE = N_PAD // NS


_DNUM_T = (((0,), (0,)), ((), ()))


def _edge_term_body(eft_ref, tet_ref, we_ref, wt_ref, o_ref):
    o_ref[...] = (
        lax.dot_general(eft_ref[...], we_ref[...], _DNUM_T,
                        preferred_element_type=jnp.float32)
        + lax.dot_general(tet_ref[...], wt_ref[...], _DNUM_T,
                          preferred_element_type=jnp.float32)
    )


def _edge_term(eft, tet, we, wt):
    blk = 6400
    grid = N_EDGES // blk
    return pl.pallas_call(
        _edge_term_body,
        grid=(grid,),
        in_specs=[
            pl.BlockSpec((16, blk), lambda i: (0, i)),
            pl.BlockSpec((16, blk), lambda i: (0, i)),
            pl.BlockSpec((16, DM), lambda i: (0, 0)),
            pl.BlockSpec((16, DM), lambda i: (0, 0)),
        ],
        out_specs=pl.BlockSpec((blk, DM), lambda i: (i, 0)),
        out_shape=jax.ShapeDtypeStruct((N_EDGES, DM), jnp.float32),
    )(eft, tet, we, wt)


def _node_term_body(mem_ref, wm_ref, q_ref, oa_ref, ob_ref):
    r = (
        jnp.dot(mem_ref[...], wm_ref[...], preferred_element_type=jnp.float32)
        + q_ref[...]
    )
    oa_ref[...] = r[:, :DH]
    ob_ref[...] = r[:, DH:]


def _node_term(mem, wm, q):
    return pl.pallas_call(
        _node_term_body,
        grid=(1,),
        in_specs=[
            pl.BlockSpec((N_NODES, DM), lambda i: (0, 0)),
            pl.BlockSpec((DM, DM), lambda i: (0, 0)),
            pl.BlockSpec((N_NODES, DM), lambda i: (0, 0)),
        ],
        out_specs=[
            pl.BlockSpec((N_NODES, DH), lambda i: (0, 0)),
            pl.BlockSpec((N_NODES, DH), lambda i: (0, 0)),
        ],
        out_shape=[
            jax.ShapeDtypeStruct((N_NODES, DH), jnp.float32),
            jax.ShapeDtypeStruct((N_NODES, DH), jnp.float32),
        ],
    )(mem, wm, q)


def _q_body(nf_ref, wf_ref, o_ref):
    o_ref[...] = jnp.dot(nf_ref[...], wf_ref[...], preferred_element_type=jnp.float32)


def _q_term(nf, wf):
    return pl.pallas_call(
        _q_body,
        grid=(1,),
        in_specs=[
            pl.BlockSpec((N_NODES, DM), lambda i: (0, 0)),
            pl.BlockSpec((DM, DM), lambda i: (0, 0)),
        ],
        out_specs=pl.BlockSpec((N_NODES, DM), lambda i: (0, 0)),
        out_shape=jax.ShapeDtypeStruct((N_NODES, DM), jnp.float32),
    )(nf, wf)


def _update_body(agga_ref, aggb_ref, mem_ref, wu_ref, uu_ref, b_ref, m1_ref,
                 o_ref, *, mean_with_m1):
    ha = agga_ref[0, :N_NODES] + agga_ref[1, :N_NODES]
    hb = aggb_ref[0, :N_NODES] + aggb_ref[1, :N_NODES]
    agg = jnp.concatenate([ha, hb], axis=-1)
    mem2 = jnp.tanh(
        jnp.dot(agg, wu_ref[...], preferred_element_type=jnp.float32)
        + jnp.dot(mem_ref[...], uu_ref[...], preferred_element_type=jnp.float32)
        + b_ref[...]
    )
    if mean_with_m1:
        o_ref[...] = 0.5 * (m1_ref[...] + mem2)
    else:
        o_ref[...] = mem2


def _update(agga, aggb, mem, wu, uu, b, m1, mean_with_m1):
    return pl.pallas_call(
        functools.partial(_update_body, mean_with_m1=mean_with_m1),
        grid=(1,),
        in_specs=[
            pl.BlockSpec((NC, N_PAD, DH), lambda i: (0, 0, 0)),
            pl.BlockSpec((NC, N_PAD, DH), lambda i: (0, 0, 0)),
            pl.BlockSpec((N_NODES, DM), lambda i: (0, 0)),
            pl.BlockSpec((DM, DM), lambda i: (0, 0)),
            pl.BlockSpec((DM, DM), lambda i: (0, 0)),
            pl.BlockSpec((1, DM), lambda i: (0, 0)),
            pl.BlockSpec((N_NODES, DM), lambda i: (0, 0)),
        ],
        out_specs=pl.BlockSpec((N_NODES, DM), lambda i: (0, 0)),
        out_shape=jax.ShapeDtypeStruct((N_NODES, DM), jnp.float32),
    )(agga, aggb, mem, wu, uu, b, m1)


def _sc_body(p_hbm, e_hbm, src_hbm, dst_hbm, zeros_hbm, out_hbm, *refs, hoff):
    rows = refs[0:NB]
    evs = refs[NB:2 * NB]
    src_v = refs[2 * NB]
    dst_v = refs[2 * NB + 1]
    agg_sh = refs[2 * NB + 2]
    g_sems = refs[2 * NB + 3:2 * NB + 3 + NB]
    e_sems = refs[2 * NB + 3 + NB:2 * NB + 3 + 2 * NB]
    s_sems = refs[2 * NB + 3 + 2 * NB:2 * NB + 3 + 3 * NB]

    c = lax.axis_index("c")
    s = lax.axis_index("s")
    wid = c * NS + s

    pltpu.sync_copy(zeros_hbm, agg_sh.at[pl.ds(s * ROWS_PER_TILE, ROWS_PER_TILE)])
    plsc.subcore_barrier()

    base_edge = wid * EDGES_PER_W
    pltpu.sync_copy(src_hbm.at[wid], src_v)
    pltpu.sync_copy(dst_hbm.at[wid], dst_v)

    def issue(g, b):
        j = g * NB + b
        pltpu.async_copy(p_hbm.at[src_v.at[j]], rows[b], g_sems[b])
        pltpu.async_copy(
            e_hbm.at[pl.ds(base_edge + j * CHUNK, CHUNK), pl.ds(hoff, DH)],
            evs[b], e_sems[b])

    for b in range(NB):
        issue(0, b)

    def group(g, carry):
        for b in range(NB):
            j = g * NB + b
            pltpu.make_async_copy(p_hbm.at[src_v.at[j]], rows[b], g_sems[b]).wait()
            pltpu.make_async_copy(
                e_hbm.at[pl.ds(base_edge + j * CHUNK, CHUNK), pl.ds(hoff, DH)],
                evs[b], e_sems[b]).wait()

            def erow(r, carry2, b=b):
                for k in range(DH // 16):
                    sl = pl.ds(k * 16, 16)
                    rows[b][r, sl] = jnp.maximum(rows[b][r, sl] + evs[b][r, sl], 0.0)
                return carry2

            lax.fori_loop(0, CHUNK, erow, 0)

            pltpu.async_copy(rows[b], agg_sh.at[dst_v.at[j]], s_sems[b], add=True)

        @pl.when(g < GROUPS - 1)
        def _prefetch():
            for b in range(NB):
                j = g * NB + b
                pltpu.make_async_copy(
                    rows[b], agg_sh.at[dst_v.at[j]], s_sems[b]).wait()
                issue(g + 1, b)

        return carry

    lax.fori_loop(0, GROUPS, group, 0)

    for b in range(NB):
        j = (GROUPS - 1) * NB + b
        pltpu.make_async_copy(rows[b], agg_sh.at[dst_v.at[j]], s_sems[b]).wait()

    plsc.subcore_barrier()

    pltpu.sync_copy(
        agg_sh.at[pl.ds(s * ROWS_PER_TILE, ROWS_PER_TILE)],
        out_hbm.at[c, pl.ds(s * ROWS_PER_TILE, ROWS_PER_TILE)],
    )


def _make_sc_pass(hoff):
    return functools.partial(
        pl.kernel,
        out_type=jax.ShapeDtypeStruct((NC, N_PAD, DH), jnp.float32),
        mesh=plsc.VectorSubcoreMesh(core_axis_name="c", subcore_axis_name="s",
                                    num_cores=NC, num_subcores=NS),
        scratch_types=(
            [pltpu.VMEM((CHUNK, DH), jnp.float32) for _ in range(2 * NB)]
            + [
                pltpu.VMEM((STEPS, CHUNK), jnp.int32),
                pltpu.VMEM((STEPS, CHUNK), jnp.int32),
                pltpu.VMEM_SHARED((N_PAD, DH), jnp.float32),
            ]
            + [pltpu.SemaphoreType.DMA for _ in range(3 * NB)]
        ),
        compiler_params=pltpu.CompilerParams(use_tc_tiling_on_sc=False),
    )(functools.partial(_sc_body, hoff=hoff))


_sc_edge_pass_a = _make_sc_pass(0)
_sc_edge_pass_b = _make_sc_pass(DH)


def kernel(node_memory, node_features, edge_features, time_encoding,
           edge_index_causal, edge_index_conseq,
           W_msg, W_upd, U_upd, b_upd):
    wm = W_msg[:DM]
    wf = W_msg[DM:2 * DM]
    wet = W_msg[2 * DM:]

    we = wet[:16]
    wt = wet[16:]
    e_term = _edge_term(edge_features.T, time_encoding.T, we, wt)
    q = _q_term(node_features, wf)

    zeros = jnp.zeros((ROWS_PER_TILE, DH), jnp.float32)

    b2 = b_upd.reshape(1, DM)

    src1 = edge_index_causal[0].astype(jnp.int32).reshape(NW, STEPS, CHUNK)
    dst1 = edge_index_causal[1].astype(jnp.int32).reshape(NW, STEPS, CHUNK)
    src2 = edge_index_conseq[0].astype(jnp.int32).reshape(NW, STEPS, CHUNK)
    dst2 = edge_index_conseq[1].astype(jnp.int32).reshape(NW, STEPS, CHUNK)

    p1a, p1b = _node_term(node_memory, wm, q)
    agg1a = _sc_edge_pass_a(p1a, e_term, src1, dst1, zeros)
    agg1b = _sc_edge_pass_b(p1b, e_term, src1, dst1, zeros)
    mem1 = _update(agg1a, agg1b, node_memory, W_upd, U_upd, b2, node_memory, False)

    p2a, p2b = _node_term(mem1, wm, q)
    agg2a = _sc_edge_pass_a(p2a, e_term, src2, dst2, zeros)
    agg2b = _sc_edge_pass_b(p2b, e_term, src2, dst2, zeros)
    return _update(agg2a, agg2b, mem1, W_upd, U_upd, b2, mem1, True)
